# baseline (speedup 1.0000x reference)
import functools

import jax
import jax.numpy as jnp
from jax import lax
from jax.experimental import pallas as pl
from jax.experimental.pallas import tpu as pltpu
from jax.experimental.pallas import tpu_sc as plsc

_B, _T, _D = 1024, 50, 300
_TC_ROWS = 512
_SC_ROWS = _B - _TC_ROWS
_CHUNK = 2
_BLOCK_B = 64


def _sc_copy(v_hbm, o_hbm, buf0, buf1, sem_in, sem_out, nc):
    wid = lax.axis_index("s") * nc + lax.axis_index("c")
    rows_per_w = _SC_ROWS // (nc * 16)
    n = rows_per_w // _CHUNK
    base = _TC_ROWS + wid * rows_per_w
    obase = wid * rows_per_w
    bufs = (buf0, buf1)

    ins = [
        pltpu.make_async_copy(
            v_hbm.at[pl.ds(base + i * _CHUNK, _CHUNK)], bufs[i % 2], sem_in.at[i % 2]
        )
        for i in range(n)
    ]
    outs = [
        pltpu.make_async_copy(
            bufs[i % 2], o_hbm.at[pl.ds(obase + i * _CHUNK, _CHUNK)], sem_out.at[i % 2]
        )
        for i in range(n)
    ]

    ins[0].start()
    for i in range(n):
        ins[i].wait()
        outs[i].start()
        if i + 1 < n:
            if i >= 1:
                outs[i - 1].wait()
            ins[i + 1].start()
    if n >= 2:
        outs[n - 2].wait()
    outs[n - 1].wait()


def _tc_copy_block(v_ref, o_ref):
    o_ref[...] = v_ref[...]


def kernel(video, ques, attr, emb):
    del ques, attr, emb
    info = plsc.get_sparse_core_info()
    nc = info.num_cores
    mesh = plsc.VectorSubcoreMesh(core_axis_name="c", subcore_axis_name="s")
    sc_k = pl.kernel(
        functools.partial(_sc_copy, nc=nc),
        out_type=jax.ShapeDtypeStruct((_SC_ROWS, _T, _D), jnp.float32),
        mesh=mesh,
        scratch_types=[
            pltpu.VMEM((_CHUNK, _T, _D), jnp.float32),
            pltpu.VMEM((_CHUNK, _T, _D), jnp.float32),
            pltpu.SemaphoreType.DMA((2,)),
            pltpu.SemaphoreType.DMA((2,)),
        ],
    )
    back = sc_k(video)

    front = pl.pallas_call(
        _tc_copy_block,
        grid=(_TC_ROWS // _BLOCK_B,),
        in_specs=[pl.BlockSpec((_BLOCK_B, _T, _D), lambda i: (i, 0, 0))],
        out_specs=pl.BlockSpec((_BLOCK_B, _T, _D), lambda i: (i, 0, 0)),
        out_shape=jax.ShapeDtypeStruct((_TC_ROWS, _T, _D), video.dtype),
    )(video)

    return jnp.concatenate([front, back], axis=0)


# speedup vs baseline: 1.1415x; 1.1415x over previous
import functools

import jax
import jax.numpy as jnp
from jax import lax
from jax.experimental import pallas as pl
from jax.experimental.pallas import tpu as pltpu
from jax.experimental.pallas import tpu_sc as plsc

_B, _T, _D = 1024, 50, 300
_CHUNK = 2


def _sc_copy(v_hbm, o_hbm, buf0, buf1, sem_in, sem_out, nc):
    wid = lax.axis_index("s") * nc + lax.axis_index("c")
    rows_per_w = _B // (nc * 16)
    n = rows_per_w // _CHUNK
    base = wid * rows_per_w
    bufs = (buf0, buf1)

    ins = [
        pltpu.make_async_copy(
            v_hbm.at[pl.ds(base + i * _CHUNK, _CHUNK)], bufs[i % 2], sem_in.at[i % 2]
        )
        for i in range(n)
    ]
    outs = [
        pltpu.make_async_copy(
            bufs[i % 2], o_hbm.at[pl.ds(base + i * _CHUNK, _CHUNK)], sem_out.at[i % 2]
        )
        for i in range(n)
    ]

    ins[0].start()
    for i in range(n):
        ins[i].wait()
        outs[i].start()
        if i + 1 < n:
            if i >= 1:
                outs[i - 1].wait()
            ins[i + 1].start()
    if n >= 2:
        outs[n - 2].wait()
    outs[n - 1].wait()


def kernel(video, ques, attr, emb):
    del ques, attr, emb
    info = plsc.get_sparse_core_info()
    nc = info.num_cores
    mesh = plsc.VectorSubcoreMesh(core_axis_name="c", subcore_axis_name="s")
    k = pl.kernel(
        functools.partial(_sc_copy, nc=nc),
        out_type=jax.ShapeDtypeStruct((_B, _T, _D), jnp.float32),
        mesh=mesh,
        scratch_types=[
            pltpu.VMEM((_CHUNK, _T, _D), jnp.float32),
            pltpu.VMEM((_CHUNK, _T, _D), jnp.float32),
            pltpu.SemaphoreType.DMA((2,)),
            pltpu.SemaphoreType.DMA((2,)),
        ],
    )
    return k(video)


# speedup vs baseline: 1.2787x; 1.1202x over previous
import jax
import jax.numpy as jnp
from jax.experimental import pallas as pl
from jax.experimental.pallas import tpu as pltpu

_B, _T, _D = 1024, 50, 300
_CHUNK = 64
_NBUF = 4
_AHEAD = 3


def _copy_ring(v_ref, o_ref, b0, b1, b2, b3, sem_in, sem_out):
    bufs = (b0, b1, b2, b3)
    n = _B // _CHUNK

    ins = [
        pltpu.make_async_copy(
            v_ref.at[pl.ds(i * _CHUNK, _CHUNK)],
            bufs[i % _NBUF],
            sem_in.at[i % _NBUF],
        )
        for i in range(n)
    ]
    outs = [
        pltpu.make_async_copy(
            bufs[i % _NBUF],
            o_ref.at[pl.ds(i * _CHUNK, _CHUNK)],
            sem_out.at[i % _NBUF],
        )
        for i in range(n)
    ]

    waited = [False] * n
    for i in range(_AHEAD):
        ins[i].start()
    for i in range(n):
        ins[i].wait()
        outs[i].start()
        j = i + _AHEAD
        if j < n:
            k = j - _NBUF
            if k >= 0:
                outs[k].wait()
                waited[k] = True
            ins[j].start()
    for i in range(n):
        if not waited[i]:
            outs[i].wait()


def kernel(video, ques, attr, emb):
    del ques, attr, emb
    out = pl.pallas_call(
        _copy_ring,
        in_specs=[pl.BlockSpec(memory_space=pl.ANY)],
        out_specs=pl.BlockSpec(memory_space=pl.ANY),
        out_shape=jax.ShapeDtypeStruct((_B, _T, _D), video.dtype),
        scratch_shapes=[
            pltpu.VMEM((_CHUNK, _T, _D), jnp.float32),
            pltpu.VMEM((_CHUNK, _T, _D), jnp.float32),
            pltpu.VMEM((_CHUNK, _T, _D), jnp.float32),
            pltpu.VMEM((_CHUNK, _T, _D), jnp.float32),
            pltpu.SemaphoreType.DMA((_NBUF,)),
            pltpu.SemaphoreType.DMA((_NBUF,)),
        ],
    )(video)
    return out
